# baseline (speedup 1.0000x reference)
import jax
import jax.numpy as jnp
from jax.experimental import pallas as pl
from jax.experimental.pallas import tpu as pltpu

N = 208
NFEAT = 512
NHID = 256
NCLASS = 128
NCHUNK = 8
ROWS = N // NCHUNK
CHUNK = ROWS * NCLASS


def _fused(x_hbm, adj_hbm, w1_hbm, b1_ref, w2_hbm, b2_ref,
           fc1w_ref, fc1b_ref, fc2w_ref, fc2b_ref, out_ref,
           xv, adjv, w1v, w2v, h2s, acc_ref, in_sem):
    i = pl.program_id(0)

    @pl.when(i == 0)
    def _gcn():
        cp_x = pltpu.make_async_copy(x_hbm, xv, in_sem.at[0])
        cp_adj = pltpu.make_async_copy(adj_hbm, adjv, in_sem.at[1])
        cp_w1 = pltpu.make_async_copy(w1_hbm, w1v, in_sem.at[2])
        cp_w2 = pltpu.make_async_copy(w2_hbm, w2v, in_sem.at[3])
        cp_x.start()
        cp_w1.start()
        cp_adj.start()
        cp_w2.start()
        cp_x.wait()
        cp_w1.wait()
        x_ = xv[...]
        t1a = jnp.dot(x_, w1v[:, :NHID // 2],
                      preferred_element_type=jnp.float32)
        t1b = jnp.dot(x_, w1v[:, NHID // 2:],
                      preferred_element_type=jnp.float32)
        cp_adj.wait()
        adj = adjv[...]
        h1a = jnp.maximum(
            jnp.dot(adj, t1a, preferred_element_type=jnp.float32)
            + b1_ref[:, :NHID // 2], 0.0)
        h1b = jnp.maximum(
            jnp.dot(adj, t1b, preferred_element_type=jnp.float32)
            + b1_ref[:, NHID // 2:], 0.0)
        cp_w2.wait()
        t2 = (jnp.dot(h1a, w2v[:NHID // 2],
                      preferred_element_type=jnp.float32)
              + jnp.dot(h1b, w2v[NHID // 2:],
                        preferred_element_type=jnp.float32))
        h2 = jnp.maximum(jnp.dot(adj, t2, preferred_element_type=jnp.float32)
                         + b2_ref[...], 0.0)
        h2s[...] = h2.reshape(NCHUNK, ROWS, NCLASS)
        acc_ref[...] = jnp.zeros((1, 60), jnp.float32)

    @pl.when(i > 0)
    def _chunk():
        flat = h2s[i - 1].reshape(1, CHUNK)
        acc_ref[...] += jax.lax.dot_general(
            flat, fc1w_ref[...], (((1,), (1,)), ((), ())),
            preferred_element_type=jnp.float32)

    @pl.when(i == NCHUNK)
    def _head():
        h3 = jnp.maximum(acc_ref[...] + fc1b_ref[...], 0.0)
        z = jnp.sum(h3 * fc2w_ref[...], axis=1, keepdims=True)
        out_ref[...] = jax.nn.sigmoid(z + fc2b_ref[0, 0])


def kernel(x, adj, W1, b1, W2, b2, fc1_w, fc1_b, fc2_w, fc2_b):
    out = pl.pallas_call(
        _fused,
        grid=(NCHUNK + 1,),
        in_specs=[
            pl.BlockSpec(memory_space=pl.ANY),
            pl.BlockSpec(memory_space=pl.ANY),
            pl.BlockSpec(memory_space=pl.ANY),
            pl.BlockSpec((1, NHID), lambda i: (0, 0)),
            pl.BlockSpec(memory_space=pl.ANY),
            pl.BlockSpec((1, NCLASS), lambda i: (0, 0)),
            pl.BlockSpec((60, CHUNK), lambda i: (0, jnp.maximum(i - 1, 0))),
            pl.BlockSpec((1, 60), lambda i: (0, 0)),
            pl.BlockSpec((1, 60), lambda i: (0, 0)),
            pl.BlockSpec(memory_space=pltpu.SMEM),
        ],
        out_specs=pl.BlockSpec((1, 1), lambda i: (0, 0)),
        out_shape=jax.ShapeDtypeStruct((1, 1), jnp.float32),
        scratch_shapes=[
            pltpu.VMEM((N, NFEAT), jnp.float32),
            pltpu.VMEM((N, N), jnp.float32),
            pltpu.VMEM((NFEAT, NHID), jnp.float32),
            pltpu.VMEM((NHID, NCLASS), jnp.float32),
            pltpu.VMEM((NCHUNK, ROWS, NCLASS), jnp.float32),
            pltpu.VMEM((1, 60), jnp.float32),
            pltpu.SemaphoreType.DMA((4,)),
        ],
        compiler_params=pltpu.CompilerParams(
            dimension_semantics=("arbitrary",),
        ),
    )(x, adj, W1, b1.reshape(1, NHID), W2, b2.reshape(1, NCLASS),
      fc1_w, fc1_b.reshape(1, 60), fc2_w, fc2_b.reshape(1, 1))
    return out.reshape(1)


# speedup vs baseline: 1.6144x; 1.6144x over previous
import jax
import jax.numpy as jnp
from jax.experimental import pallas as pl
from jax.experimental.pallas import tpu as pltpu

N = 208
NFEAT = 512
NHID = 256
NCLASS = 128
NCHUNK = 1
CHUNK = (N * NCLASS) // NCHUNK


def _fused(x_hbm, adj_hbm, w1_hbm, b1_ref, w2_hbm, b2_ref,
           fc1w_hbm, fc1b_ref, fc2w_ref, fc2b_ref, out_ref,
           xv, adjv, w1v, w2v, fc1v, in_sem, fc_sem):
    cp_x = pltpu.make_async_copy(x_hbm, xv, in_sem.at[0])
    cp_adj = pltpu.make_async_copy(adj_hbm, adjv, in_sem.at[1])
    cp_w1 = pltpu.make_async_copy(w1_hbm, w1v, in_sem.at[2])
    cp_w2 = pltpu.make_async_copy(w2_hbm, w2v, in_sem.at[3])
    cp_fc = [
        pltpu.make_async_copy(
            fc1w_hbm.at[:, pl.ds(k * CHUNK, CHUNK)],
            fc1v.at[:, pl.ds(k * CHUNK, CHUNK)],
            fc_sem.at[k])
        for k in range(NCHUNK)
    ]
    cp_x.start()
    cp_w1.start()
    cp_adj.start()
    cp_w2.start()
    for cp in cp_fc:
        cp.start()

    cp_x.wait()
    cp_w1.wait()
    x_ = xv[...]
    t1a = jnp.dot(x_, w1v[:, :NHID // 2], preferred_element_type=jnp.float32)
    t1b = jnp.dot(x_, w1v[:, NHID // 2:], preferred_element_type=jnp.float32)
    cp_adj.wait()
    adj = adjv[...]
    h1a = jnp.maximum(jnp.dot(adj, t1a, preferred_element_type=jnp.float32)
                      + b1_ref[:, :NHID // 2], 0.0)
    h1b = jnp.maximum(jnp.dot(adj, t1b, preferred_element_type=jnp.float32)
                      + b1_ref[:, NHID // 2:], 0.0)
    cp_w2.wait()
    t2 = (jnp.dot(h1a, w2v[:NHID // 2], preferred_element_type=jnp.float32)
          + jnp.dot(h1b, w2v[NHID // 2:], preferred_element_type=jnp.float32))
    h2 = jnp.maximum(jnp.dot(adj, t2, preferred_element_type=jnp.float32)
                     + b2_ref[...], 0.0)
    flat = h2.reshape(1, N * NCLASS)

    h3 = jnp.zeros((1, 60), jnp.float32)
    for k in range(NCHUNK):
        cp_fc[k].wait()
        h3 = h3 + jax.lax.dot_general(
            flat[:, k * CHUNK:(k + 1) * CHUNK],
            fc1v[:, k * CHUNK:(k + 1) * CHUNK],
            (((1,), (1,)), ((), ())),
            preferred_element_type=jnp.float32)
    h3 = jnp.maximum(h3 + fc1b_ref[...], 0.0)
    z = jnp.sum(h3 * fc2w_ref[...], axis=1, keepdims=True)
    out_ref[...] = jax.nn.sigmoid(z + fc2b_ref[0, 0])


def kernel(x, adj, W1, b1, W2, b2, fc1_w, fc1_b, fc2_w, fc2_b):
    out = pl.pallas_call(
        _fused,
        out_shape=jax.ShapeDtypeStruct((1, 1), jnp.float32),
        in_specs=[
            pl.BlockSpec(memory_space=pl.ANY),
            pl.BlockSpec(memory_space=pl.ANY),
            pl.BlockSpec(memory_space=pl.ANY),
            pl.BlockSpec(memory_space=pltpu.VMEM),
            pl.BlockSpec(memory_space=pl.ANY),
            pl.BlockSpec(memory_space=pltpu.VMEM),
            pl.BlockSpec(memory_space=pl.ANY),
            pl.BlockSpec(memory_space=pltpu.VMEM),
            pl.BlockSpec(memory_space=pltpu.VMEM),
            pl.BlockSpec(memory_space=pltpu.SMEM),
        ],
        out_specs=pl.BlockSpec(memory_space=pltpu.VMEM),
        scratch_shapes=[
            pltpu.VMEM((N, NFEAT), jnp.float32),
            pltpu.VMEM((N, N), jnp.float32),
            pltpu.VMEM((NFEAT, NHID), jnp.float32),
            pltpu.VMEM((NHID, NCLASS), jnp.float32),
            pltpu.VMEM((60, N * NCLASS), jnp.float32),
            pltpu.SemaphoreType.DMA((4,)),
            pltpu.SemaphoreType.DMA((NCHUNK,)),
        ],
    )(x, adj, W1, b1.reshape(1, NHID), W2, b2.reshape(1, NCLASS),
      fc1_w, fc1_b.reshape(1, 60), fc2_w, fc2_b.reshape(1, 1))
    return out.reshape(1)


# speedup vs baseline: 1.6674x; 1.0328x over previous
import jax
import jax.numpy as jnp
from jax.experimental import pallas as pl
from jax.experimental.pallas import tpu as pltpu

N = 208
NFEAT = 512
NHID = 256
NCLASS = 128
NCHUNK = 2
CHUNK = (N * NCLASS) // NCHUNK


def _fused(x_hbm, adj_hbm, w1_hbm, b1_ref, w2_hbm, b2_ref,
           fc1w_hbm, fc1b_ref, fc2w_ref, fc2b_ref, out_ref,
           xv, adjv, w1v, w2v, fc1v, in_sem, fc_sem):
    cp_x = pltpu.make_async_copy(x_hbm, xv, in_sem.at[0])
    cp_adj = pltpu.make_async_copy(adj_hbm, adjv, in_sem.at[1])
    cp_w1 = pltpu.make_async_copy(w1_hbm, w1v, in_sem.at[2])
    cp_w2 = pltpu.make_async_copy(w2_hbm, w2v, in_sem.at[3])
    cp_fc = [
        pltpu.make_async_copy(
            fc1w_hbm.at[:, pl.ds(k * CHUNK, CHUNK)],
            fc1v.at[:, pl.ds(k * CHUNK, CHUNK)],
            fc_sem.at[k])
        for k in range(NCHUNK)
    ]
    cp_x.start()
    cp_w1.start()
    cp_adj.start()
    cp_w2.start()
    for cp in cp_fc:
        cp.start()

    cp_x.wait()
    cp_w1.wait()
    x_ = xv[...]
    t1a = jnp.dot(x_, w1v[:, :NHID // 2], preferred_element_type=jnp.float32)
    t1b = jnp.dot(x_, w1v[:, NHID // 2:], preferred_element_type=jnp.float32)
    cp_adj.wait()
    adj = adjv[...]
    h1a = jnp.maximum(jnp.dot(adj, t1a, preferred_element_type=jnp.float32)
                      + b1_ref[:, :NHID // 2], 0.0)
    h1b = jnp.maximum(jnp.dot(adj, t1b, preferred_element_type=jnp.float32)
                      + b1_ref[:, NHID // 2:], 0.0)
    cp_w2.wait()
    t2 = (jnp.dot(h1a, w2v[:NHID // 2], preferred_element_type=jnp.float32)
          + jnp.dot(h1b, w2v[NHID // 2:], preferred_element_type=jnp.float32))
    h2 = jnp.maximum(jnp.dot(adj, t2, preferred_element_type=jnp.float32)
                     + b2_ref[...], 0.0)
    flat = h2.reshape(1, N * NCLASS)

    h3 = jnp.zeros((1, 60), jnp.float32)
    for k in range(NCHUNK):
        cp_fc[k].wait()
        h3 = h3 + jax.lax.dot_general(
            flat[:, k * CHUNK:(k + 1) * CHUNK],
            fc1v[:, k * CHUNK:(k + 1) * CHUNK],
            (((1,), (1,)), ((), ())),
            preferred_element_type=jnp.float32)
    h3 = jnp.maximum(h3 + fc1b_ref[...], 0.0)
    z = jnp.sum(h3 * fc2w_ref[...], axis=1, keepdims=True)
    out_ref[...] = jax.nn.sigmoid(z + fc2b_ref[0, 0])


def kernel(x, adj, W1, b1, W2, b2, fc1_w, fc1_b, fc2_w, fc2_b):
    out = pl.pallas_call(
        _fused,
        out_shape=jax.ShapeDtypeStruct((1, 1), jnp.float32),
        in_specs=[
            pl.BlockSpec(memory_space=pl.ANY),
            pl.BlockSpec(memory_space=pl.ANY),
            pl.BlockSpec(memory_space=pl.ANY),
            pl.BlockSpec(memory_space=pltpu.VMEM),
            pl.BlockSpec(memory_space=pl.ANY),
            pl.BlockSpec(memory_space=pltpu.VMEM),
            pl.BlockSpec(memory_space=pl.ANY),
            pl.BlockSpec(memory_space=pltpu.VMEM),
            pl.BlockSpec(memory_space=pltpu.VMEM),
            pl.BlockSpec(memory_space=pltpu.SMEM),
        ],
        out_specs=pl.BlockSpec(memory_space=pltpu.VMEM),
        scratch_shapes=[
            pltpu.VMEM((N, NFEAT), jnp.float32),
            pltpu.VMEM((N, N), jnp.float32),
            pltpu.VMEM((NFEAT, NHID), jnp.float32),
            pltpu.VMEM((NHID, NCLASS), jnp.float32),
            pltpu.VMEM((60, N * NCLASS), jnp.float32),
            pltpu.SemaphoreType.DMA((4,)),
            pltpu.SemaphoreType.DMA((NCHUNK,)),
        ],
    )(x, adj, W1, b1.reshape(1, NHID), W2, b2.reshape(1, NCLASS),
      fc1_w, fc1_b.reshape(1, 60), fc2_w, fc2_b.reshape(1, 1))
    return out.reshape(1)
